# Initial kernel scaffold; baseline (speedup 1.0000x reference)
import jax
import jax.numpy as jnp
from jax.experimental import pallas as pl


def kernel(x, edge_index, W1, b1, gamma1, beta1, W2, b2, gamma2, beta2):
    raise NotImplementedError("write your pallas kernel here")


# speedup vs baseline: 13.1768x; 13.1768x over previous
import functools

import jax
import jax.numpy as jnp
from jax import lax
from jax.experimental import pallas as pl
from jax.experimental.pallas import tpu as pltpu
from jax.experimental.pallas import tpu_sc as plsc

N = 10000
E = 320000
D_IN = 128
D_H = 64
D_PAD = 128
EPS = 1e-5

NC = 2
NS = 16
NW = NC * NS
EPW = E // NW
CH = 80
NCH = EPW // CH
NPAD = 10240
STRIPE = NPAD // NS

_mesh = plsc.VectorSubcoreMesh(core_axis_name="c", subcore_axis_name="s")


def _zero_vec(ref, nwords):
    def body(i, _):
        ref[pl.ds(i * 16, 16)] = jnp.zeros((16,), jnp.float32)
        return 0
    lax.fori_loop(0, nwords // 16, body, 0, unroll=4)


def _zero_rows(ref, nrows, ncols):
    def body(i, _):
        for c in range(ncols // 16):
            ref[i, pl.ds(c * 16, 16)] = jnp.zeros((16,), jnp.float32)
        return 0
    lax.fori_loop(0, nrows, body, 0, unroll=2)


@functools.partial(
    pl.kernel,
    mesh=_mesh,
    out_type=jax.ShapeDtypeStruct((NC, NPAD), jnp.float32),
    scratch_types=[
        pltpu.VMEM((CH,), jnp.int32),
        pltpu.VMEM((CH,), jnp.float32),
        pltpu.VMEM((STRIPE,), jnp.float32),
        pltpu.VMEM_SHARED((NPAD,), jnp.float32),
    ],
)
def _deg(dst_hbm, out_hbm, idx_v, ones_v, zrow_v, deg_sh):
    c = lax.axis_index("c")
    s = lax.axis_index("s")
    wid = s * NC + c

    def fill(i, _):
        ones_v[pl.ds(i * 16, 16)] = jnp.ones((16,), jnp.float32)
        return 0
    lax.fori_loop(0, CH // 16, fill, 0)
    _zero_vec(zrow_v, STRIPE)

    pltpu.sync_copy(zrow_v, deg_sh.at[pl.ds(s * STRIPE, STRIPE)])
    plsc.subcore_barrier()

    def chunk(j, _):
        base = wid * EPW + j * CH
        pltpu.sync_copy(dst_hbm.at[pl.ds(base, CH)], idx_v)
        pltpu.sync_copy(ones_v, deg_sh.at[idx_v], add=True)
        return 0
    lax.fori_loop(0, NCH, chunk, 0)
    plsc.subcore_barrier()

    pltpu.sync_copy(deg_sh.at[pl.ds(s * STRIPE, STRIPE)],
                    out_hbm.at[c, pl.ds(s * STRIPE, STRIPE)])


@functools.partial(
    pl.kernel,
    mesh=_mesh,
    out_type=jax.ShapeDtypeStruct((NC, NPAD, D_PAD), jnp.float32),
    scratch_types=[
        pltpu.VMEM((CH,), jnp.int32),
        pltpu.VMEM((CH,), jnp.int32),
        pltpu.VMEM((CH, D_PAD), jnp.float32),
        pltpu.VMEM_SHARED((NPAD, D_PAD), jnp.float32),
        pltpu.SemaphoreType.DMA,
    ],
)
def _scat(src_hbm, dst_hbm, h_hbm, out_hbm, sidx_v, didx_v, rows_v, acc_sh, sem):
    c = lax.axis_index("c")
    s = lax.axis_index("s")
    wid = s * NC + c

    _zero_rows(rows_v, CH, D_PAD)
    lo = s * STRIPE

    def zc(k, _):
        pltpu.sync_copy(rows_v, acc_sh.at[pl.ds(lo + k * CH, CH)])
        return 0
    lax.fori_loop(0, STRIPE // CH, zc, 0)
    plsc.subcore_barrier()

    def chunk(j, _):
        base = wid * EPW + j * CH
        pltpu.sync_copy(src_hbm.at[pl.ds(base, CH)], sidx_v)
        pltpu.sync_copy(dst_hbm.at[pl.ds(base, CH)], didx_v)
        pltpu.async_copy(h_hbm.at[sidx_v], rows_v, sem).wait()
        pltpu.sync_copy(rows_v, acc_sh.at[didx_v], add=True)
        return 0
    lax.fori_loop(0, NCH, chunk, 0)
    plsc.subcore_barrier()

    def oc(k, _):
        pltpu.sync_copy(acc_sh.at[pl.ds(lo + k * CH, CH)],
                        out_hbm.at[c, pl.ds(lo + k * CH, CH)])
        return 0
    lax.fori_loop(0, STRIPE // CH, oc, 0)


def _dinv_of(d0, d1):
    deg = d0 + d1 + 1.0
    return lax.rsqrt(jnp.maximum(deg, 1e-12))


def _mm_body(x_ref, w_ref, d0_ref, d1_ref, o_ref):
    dinv = _dinv_of(d0_ref[...], d1_ref[...])
    o_ref[...] = dinv * jnp.dot(
        x_ref[...], w_ref[...], preferred_element_type=jnp.float32)


def _first64(ref):
    return ref[...][:, :D_H]


def _comb_body(a0, a1, hp, d0, d1, b, g, be, w2, o_ref, *, last):
    dinv = _dinv_of(d0[...], d1[...])
    t = dinv * (_first64(a0) + _first64(a1) + _first64(hp)) + b[...]
    mu = jnp.mean(t, axis=0, keepdims=True)
    tc = t - mu
    var = jnp.mean(tc * tc, axis=0, keepdims=True)
    y = jnp.maximum(tc * lax.rsqrt(var + EPS) * g[...] + be[...], 0.0)
    if last:
        o_ref[...] = y
    else:
        o_ref[...] = dinv * jnp.dot(
            y, w2[...], preferred_element_type=jnp.float32)


_mm = pl.pallas_call(
    _mm_body, out_shape=jax.ShapeDtypeStruct((N, D_PAD), jnp.float32))
_comb_mid = pl.pallas_call(
    functools.partial(_comb_body, last=False),
    out_shape=jax.ShapeDtypeStruct((N, D_PAD), jnp.float32))


def _comb_last_body(a0, a1, hp, d0, d1, b, g, be, o_ref):
    _comb_body(a0, a1, hp, d0, d1, b, g, be, None, o_ref, last=True)


_comb_last = pl.pallas_call(
    _comb_last_body, out_shape=jax.ShapeDtypeStruct((N, D_H), jnp.float32))


def kernel(x, edge_index, W1, b1, gamma1, beta1, W2, b2, gamma2, beta2):
    ei = edge_index.astype(jnp.int32)
    src = ei[0]
    dst = ei[1]

    degp = _deg(dst)
    d0 = degp[0, :N].reshape(N, 1)
    d1 = degp[1, :N].reshape(N, 1)

    W1p = jnp.pad(W1, ((0, 0), (0, D_PAD - D_H)))
    W2p = jnp.pad(W2, ((0, 0), (0, D_PAD - D_H)))

    h1p = _mm(x, W1p, d0, d1)
    accp = _scat(src, dst, h1p)
    h2p = _comb_mid(accp[0, :N], accp[1, :N], h1p, d0, d1,
                    b1.reshape(1, D_H), gamma1.reshape(1, D_H),
                    beta1.reshape(1, D_H), W2p)
    accp2 = _scat(src, dst, h2p)
    out = _comb_last(accp2[0, :N], accp2[1, :N], h2p, d0, d1,
                     b2.reshape(1, D_H), gamma2.reshape(1, D_H),
                     beta2.reshape(1, D_H))
    return out


# speedup vs baseline: 20.7828x; 1.5772x over previous
import functools

import jax
import jax.numpy as jnp
from jax import lax
from jax.experimental import pallas as pl
from jax.experimental.pallas import tpu as pltpu
from jax.experimental.pallas import tpu_sc as plsc

N = 10000
E = 320000
D_IN = 128
D_H = 64
D_PAD = 128
EPS = 1e-5

NC = 2
NS = 16
NW = NC * NS
EPW = E // NW
CH = 80
NCH = EPW // CH
NPAD = 10240
STRIPE = NPAD // NS

_mesh = plsc.VectorSubcoreMesh(core_axis_name="c", subcore_axis_name="s")


def _zero_vec(ref, nwords):
    def body(i, _):
        ref[pl.ds(i * 16, 16)] = jnp.zeros((16,), jnp.float32)
        return 0
    lax.fori_loop(0, nwords // 16, body, 0, unroll=4)


def _zero_rows(ref, nrows, ncols):
    def body(i, _):
        for c in range(ncols // 16):
            ref[i, pl.ds(c * 16, 16)] = jnp.zeros((16,), jnp.float32)
        return 0
    lax.fori_loop(0, nrows, body, 0, unroll=2)


@functools.partial(
    pl.kernel,
    mesh=_mesh,
    out_type=jax.ShapeDtypeStruct((NC, NPAD), jnp.float32),
    scratch_types=[
        pltpu.VMEM((CH,), jnp.int32),
        pltpu.VMEM((CH,), jnp.int32),
        pltpu.VMEM((CH,), jnp.float32),
        pltpu.VMEM((STRIPE,), jnp.float32),
        pltpu.VMEM_SHARED((NPAD,), jnp.float32),
        pltpu.SemaphoreType.DMA,
        pltpu.SemaphoreType.DMA,
    ],
)
def _deg(dst_hbm, out_hbm, idxa_v, idxb_v, ones_v, zrow_v, deg_sh, sema, semb):
    c = lax.axis_index("c")
    s = lax.axis_index("s")
    wid = s * NC + c
    base0 = wid * EPW

    def fill(i, _):
        ones_v[pl.ds(i * 16, 16)] = jnp.ones((16,), jnp.float32)
        return 0
    lax.fori_loop(0, CH // 16, fill, 0)
    _zero_vec(zrow_v, STRIPE)

    pltpu.sync_copy(zrow_v, deg_sh.at[pl.ds(s * STRIPE, STRIPE)])
    plsc.subcore_barrier()

    pltpu.async_copy(dst_hbm.at[pl.ds(base0, CH)], idxa_v, sema)

    def body(k, _):
        j = 2 * k
        pltpu.async_copy(dst_hbm.at[pl.ds(base0 + (j + 1) * CH, CH)], idxb_v, semb)
        pltpu.make_async_copy(dst_hbm.at[pl.ds(base0, CH)], idxa_v, sema).wait()
        pltpu.sync_copy(ones_v, deg_sh.at[idxa_v], add=True)
        pltpu.async_copy(dst_hbm.at[pl.ds(base0 + (j + 2) * CH, CH)], idxa_v, sema)
        pltpu.make_async_copy(dst_hbm.at[pl.ds(base0, CH)], idxb_v, semb).wait()
        pltpu.sync_copy(ones_v, deg_sh.at[idxb_v], add=True)
        return 0
    lax.fori_loop(0, (NCH - 1) // 2, body, 0)
    pltpu.make_async_copy(dst_hbm.at[pl.ds(base0, CH)], idxa_v, sema).wait()
    pltpu.sync_copy(ones_v, deg_sh.at[idxa_v], add=True)
    plsc.subcore_barrier()

    pltpu.sync_copy(deg_sh.at[pl.ds(s * STRIPE, STRIPE)],
                    out_hbm.at[c, pl.ds(s * STRIPE, STRIPE)])


@functools.partial(
    pl.kernel,
    mesh=_mesh,
    out_type=jax.ShapeDtypeStruct((NC, NPAD, D_PAD), jnp.float32),
    scratch_types=[
        pltpu.VMEM((CH,), jnp.int32),
        pltpu.VMEM((CH,), jnp.int32),
        pltpu.VMEM((CH, D_PAD), jnp.float32),
        pltpu.VMEM((CH,), jnp.int32),
        pltpu.VMEM((CH,), jnp.int32),
        pltpu.VMEM((CH, D_PAD), jnp.float32),
        pltpu.VMEM_SHARED((NPAD, D_PAD), jnp.float32),
        pltpu.SemaphoreType.DMA,
        pltpu.SemaphoreType.DMA,
    ],
)
def _scat(src_hbm, dst_hbm, h_hbm, out_hbm,
          sidxa_v, didxa_v, rowsa_v, sidxb_v, didxb_v, rowsb_v,
          acc_sh, sema, semb):
    c = lax.axis_index("c")
    s = lax.axis_index("s")
    wid = s * NC + c
    base0 = wid * EPW
    lo = s * STRIPE

    _zero_rows(rowsa_v, CH, D_PAD)

    def zc(k, _):
        pltpu.sync_copy(rowsa_v, acc_sh.at[pl.ds(lo + k * CH, CH)])
        return 0
    lax.fori_loop(0, STRIPE // CH, zc, 0)
    plsc.subcore_barrier()

    def load_idx(j, sidx, didx):
        pltpu.sync_copy(src_hbm.at[pl.ds(base0 + j * CH, CH)], sidx)
        pltpu.sync_copy(dst_hbm.at[pl.ds(base0 + j * CH, CH)], didx)

    load_idx(0, sidxa_v, didxa_v)
    pltpu.async_copy(h_hbm.at[sidxa_v], rowsa_v, sema)

    def body(k, _):
        j = 2 * k
        load_idx(j + 1, sidxb_v, didxb_v)
        pltpu.async_copy(h_hbm.at[sidxb_v], rowsb_v, semb)
        pltpu.make_async_copy(h_hbm.at[sidxa_v], rowsa_v, sema).wait()
        pltpu.sync_copy(rowsa_v, acc_sh.at[didxa_v], add=True)
        load_idx(j + 2, sidxa_v, didxa_v)
        pltpu.async_copy(h_hbm.at[sidxa_v], rowsa_v, sema)
        pltpu.make_async_copy(h_hbm.at[sidxb_v], rowsb_v, semb).wait()
        pltpu.sync_copy(rowsb_v, acc_sh.at[didxb_v], add=True)
        return 0
    lax.fori_loop(0, (NCH - 1) // 2, body, 0)
    pltpu.make_async_copy(h_hbm.at[sidxa_v], rowsa_v, sema).wait()
    pltpu.sync_copy(rowsa_v, acc_sh.at[didxa_v], add=True)
    plsc.subcore_barrier()

    def oc(k, _):
        pltpu.sync_copy(acc_sh.at[pl.ds(lo + k * CH, CH)],
                        out_hbm.at[c, pl.ds(lo + k * CH, CH)])
        return 0
    lax.fori_loop(0, STRIPE // CH, oc, 0)


def _dinv_of(d0, d1):
    deg = d0 + d1 + 1.0
    return lax.rsqrt(jnp.maximum(deg, 1e-12))


def _mm_body(x_ref, w_ref, o_ref):
    o_ref[...] = jnp.dot(
        x_ref[...], w_ref[...], preferred_element_type=jnp.float32)


def _scale_body(h_ref, d0_ref, d1_ref, o_ref):
    o_ref[...] = _dinv_of(d0_ref[...], d1_ref[...]) * h_ref[...]


def _first64(ref):
    return ref[...][:, :D_H]


def _comb_body(a0, a1, hp, d0, d1, b, g, be, w2, o_ref, *, last):
    dinv = _dinv_of(d0[...], d1[...])
    t = dinv * (_first64(a0) + _first64(a1) + _first64(hp)) + b[...]
    mu = jnp.mean(t, axis=0, keepdims=True)
    tc = t - mu
    var = jnp.mean(tc * tc, axis=0, keepdims=True)
    y = jnp.maximum(tc * lax.rsqrt(var + EPS) * g[...] + be[...], 0.0)
    if last:
        o_ref[...] = y
    else:
        o_ref[...] = dinv * jnp.dot(
            y, w2[...], preferred_element_type=jnp.float32)


_mm = pl.pallas_call(
    _mm_body, out_shape=jax.ShapeDtypeStruct((N, D_PAD), jnp.float32))
_scale = pl.pallas_call(
    _scale_body, out_shape=jax.ShapeDtypeStruct((N, D_PAD), jnp.float32))
_comb_mid = pl.pallas_call(
    functools.partial(_comb_body, last=False),
    out_shape=jax.ShapeDtypeStruct((N, D_PAD), jnp.float32))


def _comb_last_body(a0, a1, hp, d0, d1, b, g, be, o_ref):
    _comb_body(a0, a1, hp, d0, d1, b, g, be, None, o_ref, last=True)


_comb_last = pl.pallas_call(
    _comb_last_body, out_shape=jax.ShapeDtypeStruct((N, D_H), jnp.float32))


def kernel(x, edge_index, W1, b1, gamma1, beta1, W2, b2, gamma2, beta2):
    ei = edge_index.astype(jnp.int32)
    src = ei[0]
    dst = ei[1]

    W1p = jnp.pad(W1, ((0, 0), (0, D_PAD - D_H)))
    W2p = jnp.pad(W2, ((0, 0), (0, D_PAD - D_H)))

    h1 = _mm(x, W1p)
    degp = _deg(dst)
    d0 = degp[0, :N].reshape(N, 1)
    d1 = degp[1, :N].reshape(N, 1)
    h1p = _scale(h1, d0, d1)
    accp = _scat(src, dst, h1p)
    h2p = _comb_mid(accp[0, :N], accp[1, :N], h1p, d0, d1,
                    b1.reshape(1, D_H), gamma1.reshape(1, D_H),
                    beta1.reshape(1, D_H), W2p)
    accp2 = _scat(src, dst, h2p)
    out = _comb_last(accp2[0, :N], accp2[1, :N], h2p, d0, d1,
                     b2.reshape(1, D_H), gamma2.reshape(1, D_H),
                     beta2.reshape(1, D_H))
    return out


# speedup vs baseline: 20.8478x; 1.0031x over previous
import functools

import jax
import jax.numpy as jnp
from jax import lax
from jax.experimental import pallas as pl
from jax.experimental.pallas import tpu as pltpu
from jax.experimental.pallas import tpu_sc as plsc

N = 10000
E = 320000
D_IN = 128
D_H = 64
D_PAD = 128
EPS = 1e-5

NC = 2
NS = 16
NW = NC * NS
EPW = E // NW
CH = 80
NCH = EPW // CH
NPAD = 10240
STRIPE = NPAD // NS

_mesh = plsc.VectorSubcoreMesh(core_axis_name="c", subcore_axis_name="s")


def _zero_vec(ref, nwords):
    def body(i, _):
        ref[pl.ds(i * 16, 16)] = jnp.zeros((16,), jnp.float32)
        return 0
    lax.fori_loop(0, nwords // 16, body, 0, unroll=4)


def _zero_rows(ref, nrows, ncols):
    def body(i, _):
        for c in range(ncols // 16):
            ref[i, pl.ds(c * 16, 16)] = jnp.zeros((16,), jnp.float32)
        return 0
    lax.fori_loop(0, nrows, body, 0, unroll=2)


@functools.partial(
    pl.kernel,
    mesh=_mesh,
    out_type=jax.ShapeDtypeStruct((NC, NPAD), jnp.float32),
    scratch_types=[
        pltpu.VMEM((CH,), jnp.int32),
        pltpu.VMEM((CH,), jnp.int32),
        pltpu.VMEM((CH,), jnp.float32),
        pltpu.VMEM((STRIPE,), jnp.float32),
        pltpu.VMEM_SHARED((NPAD,), jnp.float32),
        pltpu.SemaphoreType.DMA,
        pltpu.SemaphoreType.DMA,
    ],
)
def _deg(dst_hbm, out_hbm, idxa_v, idxb_v, ones_v, zrow_v, deg_sh, sema, semb):
    c = lax.axis_index("c")
    s = lax.axis_index("s")
    wid = s * NC + c
    base0 = wid * EPW

    def fill(i, _):
        ones_v[pl.ds(i * 16, 16)] = jnp.ones((16,), jnp.float32)
        return 0
    lax.fori_loop(0, CH // 16, fill, 0)
    _zero_vec(zrow_v, STRIPE)

    pltpu.sync_copy(zrow_v, deg_sh.at[pl.ds(s * STRIPE, STRIPE)])
    plsc.subcore_barrier()

    pltpu.async_copy(dst_hbm.at[pl.ds(base0, CH)], idxa_v, sema)

    def body(k, _):
        j = 2 * k
        pltpu.async_copy(dst_hbm.at[pl.ds(base0 + (j + 1) * CH, CH)], idxb_v, semb)
        pltpu.make_async_copy(dst_hbm.at[pl.ds(base0, CH)], idxa_v, sema).wait()
        pltpu.sync_copy(ones_v, deg_sh.at[idxa_v], add=True)
        pltpu.async_copy(dst_hbm.at[pl.ds(base0 + (j + 2) * CH, CH)], idxa_v, sema)
        pltpu.make_async_copy(dst_hbm.at[pl.ds(base0, CH)], idxb_v, semb).wait()
        pltpu.sync_copy(ones_v, deg_sh.at[idxb_v], add=True)
        return 0
    lax.fori_loop(0, (NCH - 1) // 2, body, 0)
    pltpu.make_async_copy(dst_hbm.at[pl.ds(base0, CH)], idxa_v, sema).wait()
    pltpu.sync_copy(ones_v, deg_sh.at[idxa_v], add=True)
    plsc.subcore_barrier()

    pltpu.sync_copy(deg_sh.at[pl.ds(s * STRIPE, STRIPE)],
                    out_hbm.at[c, pl.ds(s * STRIPE, STRIPE)])


NBUF = 4


@functools.partial(
    pl.kernel,
    mesh=_mesh,
    out_type=jax.ShapeDtypeStruct((NC, NPAD, D_PAD), jnp.float32),
    scratch_types=(
        [pltpu.VMEM((CH,), jnp.int32) for _ in range(NBUF)]
        + [pltpu.VMEM((CH,), jnp.int32) for _ in range(NBUF)]
        + [pltpu.VMEM((CH, D_PAD), jnp.float32) for _ in range(NBUF)]
        + [pltpu.VMEM_SHARED((NPAD, D_PAD), jnp.float32)]
        + [pltpu.SemaphoreType.DMA for _ in range(NBUF)]
    ),
)
def _scat(src_hbm, dst_hbm, h_hbm, out_hbm, *refs):
    sidx = refs[0:NBUF]
    didx = refs[NBUF:2 * NBUF]
    rows = refs[2 * NBUF:3 * NBUF]
    acc_sh = refs[3 * NBUF]
    sems = refs[3 * NBUF + 1:]

    c = lax.axis_index("c")
    s = lax.axis_index("s")
    wid = s * NC + c
    base0 = wid * EPW
    lo = s * STRIPE

    _zero_rows(rows[0], CH, D_PAD)

    def zc(k, _):
        pltpu.sync_copy(rows[0], acc_sh.at[pl.ds(lo + k * CH, CH)])
        return 0
    lax.fori_loop(0, STRIPE // CH, zc, 0)
    plsc.subcore_barrier()

    def load_and_start(j, b):
        pltpu.sync_copy(src_hbm.at[pl.ds(base0 + j * CH, CH)], sidx[b])
        pltpu.sync_copy(dst_hbm.at[pl.ds(base0 + j * CH, CH)], didx[b])
        pltpu.async_copy(h_hbm.at[sidx[b]], rows[b], sems[b])

    def wait_and_scatter(b):
        pltpu.make_async_copy(h_hbm.at[sidx[b]], rows[b], sems[b]).wait()
        pltpu.sync_copy(rows[b], acc_sh.at[didx[b]], add=True)

    for b in range(NBUF - 1):
        load_and_start(b, b)

    def body(k, _):
        j0 = NBUF * k
        for b in range(NBUF):
            wait_and_scatter(b)

            @pl.when(j0 + b + NBUF - 1 < NCH)
            def _():
                load_and_start(j0 + b + NBUF - 1, (b + NBUF - 1) % NBUF)
        return 0
    lax.fori_loop(0, NCH // NBUF, body, 0)
    for j in range(NCH - NCH % NBUF, NCH):
        wait_and_scatter(j % NBUF)
    plsc.subcore_barrier()

    def oc(k, _):
        pltpu.sync_copy(acc_sh.at[pl.ds(lo + k * CH, CH)],
                        out_hbm.at[c, pl.ds(lo + k * CH, CH)])
        return 0
    lax.fori_loop(0, STRIPE // CH, oc, 0)


def _dinv_of(d0, d1):
    deg = d0 + d1 + 1.0
    return lax.rsqrt(jnp.maximum(deg, 1e-12))


def _mm_body(x_ref, w_ref, o_ref):
    o_ref[...] = jnp.dot(
        x_ref[...], w_ref[...], preferred_element_type=jnp.float32)


def _scale_body(h_ref, d0_ref, d1_ref, o_ref):
    o_ref[...] = _dinv_of(d0_ref[...], d1_ref[...]) * h_ref[...]


def _first64(ref):
    return ref[...][:, :D_H]


def _comb_body(a0, a1, hp, d0, d1, b, g, be, w2, o_ref, *, last):
    dinv = _dinv_of(d0[...], d1[...])
    t = dinv * (_first64(a0) + _first64(a1) + _first64(hp)) + b[...]
    mu = jnp.mean(t, axis=0, keepdims=True)
    tc = t - mu
    var = jnp.mean(tc * tc, axis=0, keepdims=True)
    y = jnp.maximum(tc * lax.rsqrt(var + EPS) * g[...] + be[...], 0.0)
    if last:
        o_ref[...] = y
    else:
        o_ref[...] = dinv * jnp.dot(
            y, w2[...], preferred_element_type=jnp.float32)


_mm = pl.pallas_call(
    _mm_body, out_shape=jax.ShapeDtypeStruct((N, D_PAD), jnp.float32))
_scale = pl.pallas_call(
    _scale_body, out_shape=jax.ShapeDtypeStruct((N, D_PAD), jnp.float32))
_comb_mid = pl.pallas_call(
    functools.partial(_comb_body, last=False),
    out_shape=jax.ShapeDtypeStruct((N, D_PAD), jnp.float32))


def _comb_last_body(a0, a1, hp, d0, d1, b, g, be, o_ref):
    _comb_body(a0, a1, hp, d0, d1, b, g, be, None, o_ref, last=True)


_comb_last = pl.pallas_call(
    _comb_last_body, out_shape=jax.ShapeDtypeStruct((N, D_H), jnp.float32))


def kernel(x, edge_index, W1, b1, gamma1, beta1, W2, b2, gamma2, beta2):
    ei = edge_index.astype(jnp.int32)
    src = ei[0]
    dst = ei[1]

    W1p = jnp.pad(W1, ((0, 0), (0, D_PAD - D_H)))
    W2p = jnp.pad(W2, ((0, 0), (0, D_PAD - D_H)))

    h1 = _mm(x, W1p)
    degp = _deg(dst)
    d0 = degp[0, :N].reshape(N, 1)
    d1 = degp[1, :N].reshape(N, 1)
    h1p = _scale(h1, d0, d1)
    accp = _scat(src, dst, h1p)
    h2p = _comb_mid(accp[0, :N], accp[1, :N], h1p, d0, d1,
                    b1.reshape(1, D_H), gamma1.reshape(1, D_H),
                    beta1.reshape(1, D_H), W2p)
    accp2 = _scat(src, dst, h2p)
    out = _comb_last(accp2[0, :N], accp2[1, :N], h2p, d0, d1,
                     b2.reshape(1, D_H), gamma2.reshape(1, D_H),
                     beta2.reshape(1, D_H))
    return out


# speedup vs baseline: 33.6038x; 1.6119x over previous
import functools

import jax
import jax.numpy as jnp
from jax import lax
from jax.experimental import pallas as pl
from jax.experimental.pallas import tpu as pltpu
from jax.experimental.pallas import tpu_sc as plsc

N = 10000
E = 320000
D_IN = 128
D_H = 64
D_PAD = 128
EPS = 1e-5

NC = 2
NS = 16
NW = NC * NS
EPW = E // NW
CH = 80
NCH = EPW // CH
NPAD = 10240
STRIPE = NPAD // NS

_mesh = plsc.VectorSubcoreMesh(core_axis_name="c", subcore_axis_name="s")


def _zero_vec(ref, nwords):
    def body(i, _):
        ref[pl.ds(i * 16, 16)] = jnp.zeros((16,), jnp.float32)
        return 0
    lax.fori_loop(0, nwords // 16, body, 0, unroll=4)


def _zero_rows(ref, nrows, ncols):
    def body(i, _):
        for c in range(ncols // 16):
            ref[i, pl.ds(c * 16, 16)] = jnp.zeros((16,), jnp.float32)
        return 0
    lax.fori_loop(0, nrows, body, 0, unroll=2)


@functools.partial(
    pl.kernel,
    mesh=_mesh,
    out_type=jax.ShapeDtypeStruct((NC, NPAD), jnp.float32),
    scratch_types=[
        pltpu.VMEM((CH,), jnp.int32),
        pltpu.VMEM((CH,), jnp.int32),
        pltpu.VMEM((CH,), jnp.float32),
        pltpu.VMEM((STRIPE,), jnp.float32),
        pltpu.VMEM_SHARED((NPAD,), jnp.float32),
        pltpu.SemaphoreType.DMA,
        pltpu.SemaphoreType.DMA,
    ],
)
def _deg(dst_hbm, out_hbm, idxa_v, idxb_v, ones_v, zrow_v, deg_sh, sema, semb):
    c = lax.axis_index("c")
    s = lax.axis_index("s")
    wid = s * NC + c
    base0 = wid * EPW

    def fill(i, _):
        ones_v[pl.ds(i * 16, 16)] = jnp.ones((16,), jnp.float32)
        return 0
    lax.fori_loop(0, CH // 16, fill, 0)
    _zero_vec(zrow_v, STRIPE)

    pltpu.sync_copy(zrow_v, deg_sh.at[pl.ds(s * STRIPE, STRIPE)])
    plsc.subcore_barrier()

    pltpu.async_copy(dst_hbm.at[pl.ds(base0, CH)], idxa_v, sema)

    def body(k, _):
        j = 2 * k
        pltpu.async_copy(dst_hbm.at[pl.ds(base0 + (j + 1) * CH, CH)], idxb_v, semb)
        pltpu.make_async_copy(dst_hbm.at[pl.ds(base0, CH)], idxa_v, sema).wait()
        pltpu.sync_copy(ones_v, deg_sh.at[idxa_v], add=True)
        pltpu.async_copy(dst_hbm.at[pl.ds(base0 + (j + 2) * CH, CH)], idxa_v, sema)
        pltpu.make_async_copy(dst_hbm.at[pl.ds(base0, CH)], idxb_v, semb).wait()
        pltpu.sync_copy(ones_v, deg_sh.at[idxb_v], add=True)
        return 0
    lax.fori_loop(0, (NCH - 1) // 2, body, 0)
    pltpu.make_async_copy(dst_hbm.at[pl.ds(base0, CH)], idxa_v, sema).wait()
    pltpu.sync_copy(ones_v, deg_sh.at[idxa_v], add=True)
    plsc.subcore_barrier()

    pltpu.sync_copy(deg_sh.at[pl.ds(s * STRIPE, STRIPE)],
                    out_hbm.at[c, pl.ds(s * STRIPE, STRIPE)])


NBUFI = 6
NBUFR = 4
GLEAD = 3
UNROLL = 12


@functools.partial(
    pl.kernel,
    mesh=_mesh,
    out_type=jax.ShapeDtypeStruct((NC, NPAD, D_PAD), jnp.float32),
    scratch_types=(
        [pltpu.VMEM((CH,), jnp.int32) for _ in range(NBUFI)]
        + [pltpu.VMEM((CH,), jnp.int32) for _ in range(NBUFI)]
        + [pltpu.VMEM((CH, D_PAD), jnp.float32) for _ in range(NBUFR)]
        + [pltpu.VMEM_SHARED((NPAD, D_PAD), jnp.float32)]
        + [pltpu.SemaphoreType.DMA for _ in range(NBUFI)]
        + [pltpu.SemaphoreType.DMA for _ in range(NBUFI)]
        + [pltpu.SemaphoreType.DMA for _ in range(NBUFR)]
    ),
)
def _scat(src_hbm, dst_hbm, h_hbm, out_hbm, *refs):
    sidx = refs[0:NBUFI]
    didx = refs[NBUFI:2 * NBUFI]
    rows = refs[2 * NBUFI:2 * NBUFI + NBUFR]
    acc_sh = refs[2 * NBUFI + NBUFR]
    issem = refs[2 * NBUFI + NBUFR + 1:3 * NBUFI + NBUFR + 1]
    idsem = refs[3 * NBUFI + NBUFR + 1:4 * NBUFI + NBUFR + 1]
    gsem = refs[4 * NBUFI + NBUFR + 1:]

    c = lax.axis_index("c")
    s = lax.axis_index("s")
    wid = s * NC + c
    base0 = wid * EPW
    lo = s * STRIPE

    _zero_rows(rows[0], CH, D_PAD)

    def zc(k, _):
        pltpu.sync_copy(rows[0], acc_sh.at[pl.ds(lo + k * CH, CH)])
        return 0
    lax.fori_loop(0, STRIPE // CH, zc, 0)
    plsc.subcore_barrier()

    def idx_load(j, ib):
        pltpu.async_copy(src_hbm.at[pl.ds(base0 + j * CH, CH)], sidx[ib], issem[ib])
        pltpu.async_copy(dst_hbm.at[pl.ds(base0 + j * CH, CH)], didx[ib], idsem[ib])

    def idx_wait_and_gather(j, ib, rb):
        pltpu.make_async_copy(
            src_hbm.at[pl.ds(base0 + j * CH, CH)], sidx[ib], issem[ib]).wait()
        pltpu.make_async_copy(
            dst_hbm.at[pl.ds(base0 + j * CH, CH)], didx[ib], idsem[ib]).wait()
        pltpu.async_copy(h_hbm.at[sidx[ib]], rows[rb], gsem[rb])

    def wait_and_scatter(j, ib, rb):
        pltpu.make_async_copy(h_hbm.at[sidx[ib]], rows[rb], gsem[rb]).wait()
        pltpu.sync_copy(rows[rb], acc_sh.at[didx[ib]], add=True)

    for j in range(NBUFI):
        idx_load(j, j)
    for t in range(GLEAD):
        idx_wait_and_gather(t, t, t)

    def body(k, _):
        j0 = UNROLL * k
        for u in range(UNROLL):
            j = j0 + u

            @pl.when(j < NCH)
            def _():
                wait_and_scatter(j, u % NBUFI, u % NBUFR)

                @pl.when(j + NBUFI < NCH)
                def _():
                    idx_load(j + NBUFI, u % NBUFI)

                @pl.when(j + GLEAD < NCH)
                def _():
                    idx_wait_and_gather(j + GLEAD, (u + GLEAD) % NBUFI,
                                        (u + GLEAD) % NBUFR)
        return 0
    lax.fori_loop(0, (NCH + UNROLL - 1) // UNROLL, body, 0)
    plsc.subcore_barrier()

    def oc(k, _):
        pltpu.sync_copy(acc_sh.at[pl.ds(lo + k * CH, CH)],
                        out_hbm.at[c, pl.ds(lo + k * CH, CH)])
        return 0
    lax.fori_loop(0, STRIPE // CH, oc, 0)


def _dinv_of(d0, d1):
    deg = d0 + d1 + 1.0
    return lax.rsqrt(jnp.maximum(deg, 1e-12))


def _mm_body(x_ref, w_ref, o_ref):
    o_ref[...] = jnp.dot(
        x_ref[...], w_ref[...], preferred_element_type=jnp.float32)


def _scale_body(h_ref, d0_ref, d1_ref, o_ref):
    o_ref[...] = _dinv_of(d0_ref[...], d1_ref[...]) * h_ref[...]


def _comb_body(accp, hp, d0, d1, b, g, be, w2, o_ref, *, last):
    dinv = _dinv_of(d0[...], d1[...])
    acc = accp[...]
    t = dinv * (acc[0, :N, :D_H] + acc[1, :N, :D_H]
                + hp[...][:, :D_H]) + b[...]
    mu = jnp.mean(t, axis=0, keepdims=True)
    tc = t - mu
    var = jnp.mean(tc * tc, axis=0, keepdims=True)
    y = jnp.maximum(tc * lax.rsqrt(var + EPS) * g[...] + be[...], 0.0)
    if last:
        o_ref[...] = y
    else:
        o_ref[...] = dinv * jnp.dot(
            y, w2[...], preferred_element_type=jnp.float32)


_mm = pl.pallas_call(
    _mm_body, out_shape=jax.ShapeDtypeStruct((N, D_PAD), jnp.float32))
_scale = pl.pallas_call(
    _scale_body, out_shape=jax.ShapeDtypeStruct((N, D_PAD), jnp.float32))
_comb_mid = pl.pallas_call(
    functools.partial(_comb_body, last=False),
    out_shape=jax.ShapeDtypeStruct((N, D_PAD), jnp.float32))


def _comb_last_body(accp, hp, d0, d1, b, g, be, o_ref):
    _comb_body(accp, hp, d0, d1, b, g, be, None, o_ref, last=True)


_comb_last = pl.pallas_call(
    _comb_last_body, out_shape=jax.ShapeDtypeStruct((N, D_H), jnp.float32))


def kernel(x, edge_index, W1, b1, gamma1, beta1, W2, b2, gamma2, beta2):
    ei = edge_index.astype(jnp.int32)
    src = ei[0]
    dst = ei[1]

    W1p = jnp.pad(W1, ((0, 0), (0, D_PAD - D_H)))
    W2p = jnp.pad(W2, ((0, 0), (0, D_PAD - D_H)))

    h1 = _mm(x, W1p)
    degp = _deg(dst)
    d0 = degp[0, :N].reshape(N, 1)
    d1 = degp[1, :N].reshape(N, 1)
    h1p = _scale(h1, d0, d1)
    accp = _scat(src, dst, h1p)
    h2p = _comb_mid(accp, h1p, d0, d1,
                    b1.reshape(1, D_H), gamma1.reshape(1, D_H),
                    beta1.reshape(1, D_H), W2p)
    accp2 = _scat(src, dst, h2p)
    out = _comb_last(accp2, h2p, d0, d1,
                     b2.reshape(1, D_H), gamma2.reshape(1, D_H),
                     beta2.reshape(1, D_H))
    return out


# speedup vs baseline: 34.4215x; 1.0243x over previous
import functools

import jax
import jax.numpy as jnp
from jax import lax
from jax.experimental import pallas as pl
from jax.experimental.pallas import tpu as pltpu
from jax.experimental.pallas import tpu_sc as plsc

N = 10000
E = 320000
D_IN = 128
D_H = 64
D_PAD = 128
EPS = 1e-5

NC = 2
NS = 16
NW = NC * NS
EPW = E // NW
CH = 80
NCH = EPW // CH
NPAD = 10240
STRIPE = NPAD // NS

_mesh = plsc.VectorSubcoreMesh(core_axis_name="c", subcore_axis_name="s")


def _zero_vec(ref, nwords):
    def body(i, _):
        ref[pl.ds(i * 16, 16)] = jnp.zeros((16,), jnp.float32)
        return 0
    lax.fori_loop(0, nwords // 16, body, 0, unroll=4)


def _zero_rows(ref, nrows, ncols):
    def body(i, _):
        for c in range(ncols // 16):
            ref[i, pl.ds(c * 16, 16)] = jnp.zeros((16,), jnp.float32)
        return 0
    lax.fori_loop(0, nrows, body, 0, unroll=2)


@functools.partial(
    pl.kernel,
    mesh=_mesh,
    out_type=jax.ShapeDtypeStruct((NC, NPAD), jnp.float32),
    scratch_types=[
        pltpu.VMEM((CH,), jnp.int32),
        pltpu.VMEM((CH,), jnp.int32),
        pltpu.VMEM((CH,), jnp.float32),
        pltpu.VMEM((STRIPE,), jnp.float32),
        pltpu.VMEM_SHARED((NPAD,), jnp.float32),
        pltpu.SemaphoreType.DMA,
        pltpu.SemaphoreType.DMA,
    ],
)
def _deg(dst_hbm, out_hbm, idxa_v, idxb_v, ones_v, zrow_v, deg_sh, sema, semb):
    c = lax.axis_index("c")
    s = lax.axis_index("s")
    wid = s * NC + c
    base0 = wid * EPW

    def fill(i, _):
        ones_v[pl.ds(i * 16, 16)] = jnp.ones((16,), jnp.float32)
        return 0
    lax.fori_loop(0, CH // 16, fill, 0)
    _zero_vec(zrow_v, STRIPE)

    pltpu.sync_copy(zrow_v, deg_sh.at[pl.ds(s * STRIPE, STRIPE)])
    plsc.subcore_barrier()

    pltpu.async_copy(dst_hbm.at[pl.ds(base0, CH)], idxa_v, sema)

    def body(k, _):
        j = 2 * k
        pltpu.async_copy(dst_hbm.at[pl.ds(base0 + (j + 1) * CH, CH)], idxb_v, semb)
        pltpu.make_async_copy(dst_hbm.at[pl.ds(base0, CH)], idxa_v, sema).wait()
        pltpu.sync_copy(ones_v, deg_sh.at[idxa_v], add=True)
        pltpu.async_copy(dst_hbm.at[pl.ds(base0 + (j + 2) * CH, CH)], idxa_v, sema)
        pltpu.make_async_copy(dst_hbm.at[pl.ds(base0, CH)], idxb_v, semb).wait()
        pltpu.sync_copy(ones_v, deg_sh.at[idxb_v], add=True)
        return 0
    lax.fori_loop(0, (NCH - 1) // 2, body, 0)
    pltpu.make_async_copy(dst_hbm.at[pl.ds(base0, CH)], idxa_v, sema).wait()
    pltpu.sync_copy(ones_v, deg_sh.at[idxa_v], add=True)
    plsc.subcore_barrier()

    pltpu.sync_copy(deg_sh.at[pl.ds(s * STRIPE, STRIPE)],
                    out_hbm.at[c, pl.ds(s * STRIPE, STRIPE)])


NBUFI = 6
NBUFR = 4
GLEAD = 4
UNROLL = 12


@functools.partial(
    pl.kernel,
    mesh=_mesh,
    out_type=jax.ShapeDtypeStruct((NC, NPAD, D_PAD), jnp.float32),
    scratch_types=(
        [pltpu.VMEM((CH,), jnp.int32) for _ in range(NBUFI)]
        + [pltpu.VMEM((CH,), jnp.int32) for _ in range(NBUFI)]
        + [pltpu.VMEM((CH, D_PAD), jnp.float32) for _ in range(NBUFR)]
        + [pltpu.VMEM_SHARED((NPAD, D_PAD), jnp.float32)]
        + [pltpu.SemaphoreType.DMA for _ in range(NBUFI)]
        + [pltpu.SemaphoreType.DMA for _ in range(NBUFI)]
        + [pltpu.SemaphoreType.DMA for _ in range(NBUFR)]
    ),
)
def _scat(src_hbm, dst_hbm, h_hbm, out_hbm, *refs):
    sidx = refs[0:NBUFI]
    didx = refs[NBUFI:2 * NBUFI]
    rows = refs[2 * NBUFI:2 * NBUFI + NBUFR]
    acc_sh = refs[2 * NBUFI + NBUFR]
    issem = refs[2 * NBUFI + NBUFR + 1:3 * NBUFI + NBUFR + 1]
    idsem = refs[3 * NBUFI + NBUFR + 1:4 * NBUFI + NBUFR + 1]
    gsem = refs[4 * NBUFI + NBUFR + 1:]

    c = lax.axis_index("c")
    s = lax.axis_index("s")
    wid = s * NC + c
    base0 = wid * EPW
    lo = s * STRIPE

    _zero_rows(rows[0], CH, D_PAD)

    def zc(k, _):
        pltpu.sync_copy(rows[0], acc_sh.at[pl.ds(lo + k * CH, CH)])
        return 0
    lax.fori_loop(0, STRIPE // CH, zc, 0)
    plsc.subcore_barrier()

    def idx_load(j, ib):
        pltpu.async_copy(src_hbm.at[pl.ds(base0 + j * CH, CH)], sidx[ib], issem[ib])
        pltpu.async_copy(dst_hbm.at[pl.ds(base0 + j * CH, CH)], didx[ib], idsem[ib])

    def idx_wait_and_gather(j, ib, rb):
        pltpu.make_async_copy(
            src_hbm.at[pl.ds(base0 + j * CH, CH)], sidx[ib], issem[ib]).wait()
        pltpu.make_async_copy(
            dst_hbm.at[pl.ds(base0 + j * CH, CH)], didx[ib], idsem[ib]).wait()
        pltpu.async_copy(h_hbm.at[sidx[ib]], rows[rb], gsem[rb])

    def wait_and_scatter(j, ib, rb):
        pltpu.make_async_copy(h_hbm.at[sidx[ib]], rows[rb], gsem[rb]).wait()
        pltpu.sync_copy(rows[rb], acc_sh.at[didx[ib]], add=True)

    for j in range(NBUFI):
        idx_load(j, j)
    for t in range(GLEAD):
        idx_wait_and_gather(t, t, t)

    def body(k, _):
        j0 = UNROLL * k
        for u in range(UNROLL):
            j = j0 + u

            @pl.when(j < NCH)
            def _():
                wait_and_scatter(j, u % NBUFI, u % NBUFR)

                @pl.when(j + NBUFI < NCH)
                def _():
                    idx_load(j + NBUFI, u % NBUFI)

                @pl.when(j + GLEAD < NCH)
                def _():
                    idx_wait_and_gather(j + GLEAD, (u + GLEAD) % NBUFI,
                                        (u + GLEAD) % NBUFR)
        return 0
    lax.fori_loop(0, (NCH + UNROLL - 1) // UNROLL, body, 0)
    plsc.subcore_barrier()

    def oc(k, _):
        pltpu.sync_copy(acc_sh.at[pl.ds(lo + k * CH, CH)],
                        out_hbm.at[c, pl.ds(lo + k * CH, CH)])
        return 0
    lax.fori_loop(0, STRIPE // CH, oc, 0)


def _dinv_of(d0, d1):
    deg = d0 + d1 + 1.0
    return lax.rsqrt(jnp.maximum(deg, 1e-12))


def _mm_body(x_ref, w_ref, o_ref):
    o_ref[...] = jnp.dot(
        x_ref[...], w_ref[...], preferred_element_type=jnp.float32)


def _scale_body(h_ref, d0_ref, d1_ref, o_ref):
    o_ref[...] = _dinv_of(d0_ref[...], d1_ref[...]) * h_ref[...]


def _comb_body(accp, hp, d0, d1, b, g, be, w2, o_ref, *, last):
    dinv = _dinv_of(d0[...], d1[...])
    acc = accp[...]
    t = dinv * (acc[0, :N, :D_H] + acc[1, :N, :D_H]
                + hp[...][:, :D_H]) + b[...]
    mu = jnp.mean(t, axis=0, keepdims=True)
    tc = t - mu
    var = jnp.mean(tc * tc, axis=0, keepdims=True)
    y = jnp.maximum(tc * lax.rsqrt(var + EPS) * g[...] + be[...], 0.0)
    if last:
        o_ref[...] = y
    else:
        o_ref[...] = dinv * jnp.dot(
            y, w2[...], preferred_element_type=jnp.float32)


_mm = pl.pallas_call(
    _mm_body, out_shape=jax.ShapeDtypeStruct((N, D_PAD), jnp.float32))
_scale = pl.pallas_call(
    _scale_body, out_shape=jax.ShapeDtypeStruct((N, D_PAD), jnp.float32))
_comb_mid = pl.pallas_call(
    functools.partial(_comb_body, last=False),
    out_shape=jax.ShapeDtypeStruct((N, D_PAD), jnp.float32))


def _comb_last_body(accp, hp, d0, d1, b, g, be, o_ref):
    _comb_body(accp, hp, d0, d1, b, g, be, None, o_ref, last=True)


_comb_last = pl.pallas_call(
    _comb_last_body, out_shape=jax.ShapeDtypeStruct((N, D_H), jnp.float32))


def kernel(x, edge_index, W1, b1, gamma1, beta1, W2, b2, gamma2, beta2):
    ei = edge_index.astype(jnp.int32)
    src = ei[0]
    dst = ei[1]

    W1p = jnp.pad(W1, ((0, 0), (0, D_PAD - D_H)))
    W2p = jnp.pad(W2, ((0, 0), (0, D_PAD - D_H)))

    h1 = _mm(x, W1p)
    degp = _deg(dst)
    d0 = degp[0, :N].reshape(N, 1)
    d1 = degp[1, :N].reshape(N, 1)
    h1p = _scale(h1, d0, d1)
    accp = _scat(src, dst, h1p)
    h2p = _comb_mid(accp, h1p, d0, d1,
                    b1.reshape(1, D_H), gamma1.reshape(1, D_H),
                    beta1.reshape(1, D_H), W2p)
    accp2 = _scat(src, dst, h2p)
    out = _comb_last(accp2, h2p, d0, d1,
                     b2.reshape(1, D_H), gamma2.reshape(1, D_H),
                     beta2.reshape(1, D_H))
    return out
